# baseline (speedup 1.0000x reference)
import functools

import jax
import jax.numpy as jnp
import numpy as np
from jax import lax
from jax.experimental import pallas as pl
from jax.experimental.pallas import tpu as pltpu
from jax.experimental.pallas import tpu_sc as plsc

N = 50000
G = 512
EPS = 1e-5

NP = 51200
RPT = NP // 16
E_RAW = 800000
ET = E_RAW + N
CH = 128
UB = 8
EP = 851968
NCH = EP // CH
NODE_CH = NP // CH
NODE_SLABS = NODE_CH // UB
G2 = 1024
GPT = G2 // 16
RBLK = 512


def _mesh():
    return plsc.VectorSubcoreMesh(core_axis_name="c", subcore_axis_name="s")


_SC_PARAMS = pltpu.CompilerParams(use_tc_tiling_on_sc=False)


def _on_core(c, fn):
    @pl.when(c == 0)
    def _():
        fn(0)

    @pl.when(c == 1)
    def _():
        fn(1)


def _fill(ref, rows, width, val):
    @pl.loop(0, rows)
    def _(i):
        for h in range(width // 16):
            ref[i, pl.ds(h * 16, 16)] = jnp.full((16,), val, jnp.float32)


def _sc_deg(dst_ch):
    cpt = NCH // 32

    @functools.partial(
        pl.kernel,
        out_type=jax.ShapeDtypeStruct((2, NP, 16), jnp.float32),
        mesh=_mesh(),
        compiler_params=_SC_PARAMS,
        scratch_types=[
            pltpu.VMEM_SHARED((NP, 16), jnp.float32),
            pltpu.VMEM((UB, CH), jnp.int32),
            pltpu.VMEM((CH, 16), jnp.float32),
            pltpu.VMEM((CH, 16), jnp.float32),
        ],
    )
    def k(dst, deg_out, acc, idxb, ones, bounce):
        c = lax.axis_index("c")
        s = lax.axis_index("s")
        _fill(ones, CH, 16, 1.0)
        _fill(bounce, CH, 16, 0.0)
        @pl.loop(0, RPT // CH)
        def _(j):
            pltpu.sync_copy(bounce, acc.at[pl.ds(s * RPT + j * CH, CH)])

        plsc.subcore_barrier()

        tile_c0 = (c * 16 + s) * cpt

        @pl.loop(0, cpt // UB)
        def _(i):
            pltpu.sync_copy(dst.at[pl.ds(tile_c0 + i * UB, UB)], idxb)
            for q in range(UB):
                pltpu.sync_copy(ones, acc.at[idxb.at[q]], add=True)

        plsc.subcore_barrier()

        def wout(cc):
            @pl.loop(0, RPT // CH)
            def _(j):
                r = s * RPT + j * CH
                pltpu.sync_copy(acc.at[pl.ds(r, CH)], bounce)
                pltpu.sync_copy(bounce, deg_out.at[cc, pl.ds(r, CH)])

        _on_core(c, wout)

    return k(dst_ch)


def _sc_agg(tab0_arr, tab1_arr, src_p, dst_p, width, feature_split, ub, chw):
    nch = EP // chw
    if feature_split:
        cpt = nch // 16
    else:
        cpt = nch // 32
    nit = cpt // ub
    assert nit % 2 == 0
    zr = 64
    src_ch = src_p.reshape(nch, chw)
    dst_ch = dst_p.reshape(nch, chw)

    @functools.partial(
        pl.kernel,
        out_type=jax.ShapeDtypeStruct((2, NP, width), jnp.float32),
        mesh=_mesh(),
        compiler_params=_SC_PARAMS,
        scratch_types=[
            pltpu.VMEM_SHARED((NP, width), jnp.float32),
            pltpu.VMEM((2, ub, chw), jnp.int32),
            pltpu.VMEM((2, ub, chw), jnp.int32),
            pltpu.VMEM((2, ub, chw, width), jnp.float32),
            pltpu.VMEM((zr, width), jnp.float32),
            pltpu.SemaphoreType.DMA,
        ],
    )
    def k(tab0, tab1, idx, dst, out, acc, srcb, dstb, rows, zb, sem):
        c = lax.axis_index("c")
        s = lax.axis_index("s")
        _fill(zb, zr, width, 0.0)

        @pl.loop(0, RPT // zr)
        def _(j):
            pltpu.sync_copy(zb, acc.at[pl.ds(s * RPT + j * zr, zr)])

        plsc.subcore_barrier()

        if feature_split:
            tile_c0 = s * cpt
        else:
            tile_c0 = None

        def run(cc):
            c0 = tile_c0 if feature_split else (cc * 16 + s) * cpt
            tab = tab0 if cc == 0 else tab1

            def load(b, it):
                ch0 = c0 + it * ub
                pltpu.sync_copy(idx.at[pl.ds(ch0, ub)], srcb.at[b])
                pltpu.sync_copy(dst.at[pl.ds(ch0, ub)], dstb.at[b])

            def fire(b):
                for q in range(ub):
                    pltpu.async_copy(tab.at[srcb.at[b, q]], rows.at[b, q], sem)

            def drain(b):
                for q in range(ub):
                    pltpu.make_async_copy(
                        tab.at[srcb.at[b, q]], rows.at[b, q], sem
                    ).wait()

            def scat(b):
                for q in range(ub):
                    pltpu.sync_copy(rows.at[b, q], acc.at[dstb.at[b, q]], add=True)

            load(0, 0)
            fire(0)

            @pl.loop(0, nit // 2)
            def _(i2):
                i = i2 * 2
                drain(0)
                load(1, i + 1)
                fire(1)
                scat(0)
                drain(1)

                @pl.when(i + 2 < nit)
                def _():
                    load(0, i + 2)
                    fire(0)

                scat(1)


        _on_core(c, run)
        plsc.subcore_barrier()

        def wout(cc):
            @pl.loop(0, RPT // zr)
            def _(j):
                r = s * RPT + j * zr
                pltpu.sync_copy(acc.at[pl.ds(r, zr)], zb)
                pltpu.sync_copy(zb, out.at[cc, pl.ds(r, zr)])

        _on_core(c, wout)

    return k(tab0_arr, tab1_arr, src_ch, dst_ch)


def _sc_agg_sp(t4, src_p, dst_p):
    chw = 256
    nch = EP // chw
    cpt = nch // 16
    zr = 64
    src_ch = src_p.reshape(nch, chw)
    dst_ch = dst_p.reshape(nch, chw)

    @functools.partial(
        pl.kernel,
        out_type=jax.ShapeDtypeStruct((4, NP, 16), jnp.float32),
        mesh=_mesh(),
        compiler_params=_SC_PARAMS,
        scratch_types=[
            pltpu.VMEM_SHARED((NP, 16), jnp.float32),
            pltpu.VMEM_SHARED((NP, 16), jnp.float32),
            pltpu.VMEM((2, chw), jnp.int32),
            pltpu.VMEM((2, chw), jnp.int32),
            pltpu.VMEM((2, chw, 16), jnp.float32),
            pltpu.VMEM((zr, 16), jnp.float32),
            pltpu.VMEM((zr, 16), jnp.float32),
            pltpu.SemaphoreType.DMA,
        ],
    )
    def k(tab, idx, dst, out, acc, tsp, srcb, dstb, rows, zb, bounce, sem):
        c = lax.axis_index("c")
        s = lax.axis_index("s")
        _fill(zb, zr, 16, 0.0)
        tile_c0 = s * cpt

        def load(b, it):
            ch0 = tile_c0 + it
            pltpu.sync_copy(idx.at[pl.ds(ch0, 1)], srcb.at[pl.ds(b, 1)])
            pltpu.sync_copy(dst.at[pl.ds(ch0, 1)], dstb.at[pl.ds(b, 1)])

        def fire(b):
            pltpu.async_copy(tsp.at[srcb.at[b]], rows.at[b], sem)

        def drain(b):
            pltpu.make_async_copy(tsp.at[srcb.at[b]], rows.at[b], sem).wait()

        def scat(b):
            pltpu.sync_copy(rows.at[b], acc.at[dstb.at[b]], add=True)

        def quarter(qq):
            @pl.loop(0, RPT // zr)
            def _(j):
                r = s * RPT + j * zr
                pltpu.sync_copy(zb, acc.at[pl.ds(r, zr)])
                pltpu.sync_copy(tab.at[qq, pl.ds(r, zr)], bounce)
                pltpu.sync_copy(bounce, tsp.at[pl.ds(r, zr)])

            plsc.subcore_barrier()

            load(0, 0)
            fire(0)

            @pl.loop(0, cpt // 2)
            def _(i2):
                i = i2 * 2
                drain(0)
                load(1, i + 1)
                fire(1)
                scat(0)
                drain(1)

                @pl.when(i + 2 < cpt)
                def _():
                    load(0, i + 2)
                    fire(0)

                scat(1)

            plsc.subcore_barrier()

            @pl.loop(0, RPT // zr)
            def _(j):
                r = s * RPT + j * zr
                pltpu.sync_copy(acc.at[pl.ds(r, zr)], bounce)
                pltpu.sync_copy(bounce, out.at[qq, pl.ds(r, zr)])

        def run(cc):
            quarter(2 * cc)
            quarter(2 * cc + 1)

        _on_core(c, run)

    return k(t4, src_ch, dst_ch)


def _tc_prep(degb, xp):
    def body(degb_ref, xp_ref, t0_ref, disr_ref):
        deg = degb_ref[0] + degb_ref[1]
        dis = jnp.where(deg > 0.0, lax.rsqrt(deg), 0.0)
        t0_ref[...] = dis * xp_ref[...]
        disr_ref[...] = dis

    return pl.pallas_call(
        body,
        grid=(NP // RBLK,),
        in_specs=[
            pl.BlockSpec((2, RBLK, 16), lambda i: (0, i, 0)),
            pl.BlockSpec((RBLK, 16), lambda i: (i, 0)),
        ],
        out_specs=[
            pl.BlockSpec((RBLK, 16), lambda i: (i, 0)),
            pl.BlockSpec((RBLK, 16), lambda i: (i, 0)),
        ],
        out_shape=[jax.ShapeDtypeStruct((NP, 16), jnp.float32)] * 2,
    )(degb, xp)


def _layer_y(aggb_ref, disr_ref, w_ref, b_ref, g_ref, be_ref, in_width):
    dis = disr_ref[...][:, 0:1]
    if in_width == 16:
        sx = disr_ref[...] * (aggb_ref[0] + aggb_ref[1])
    else:
        sx = dis * jnp.concatenate([aggb_ref[q] for q in range(4)], axis=1)
    z = jnp.dot(sx, w_ref[...], preferred_element_type=jnp.float32)
    alpha = g_ref[...] * np.float32(1.0 / np.sqrt(1.0 + EPS))
    y = jnp.maximum(alpha * (z + b_ref[...]) + be_ref[...], 0.0)
    return dis, y


_VEC_SPECS = [
    pl.BlockSpec((1, 64), lambda i: (0, 0)),
    pl.BlockSpec((1, 64), lambda i: (0, 0)),
    pl.BlockSpec((1, 64), lambda i: (0, 0)),
]


def _tc_layer(aggb, disr, w, b, g, be, in_width):
    nin = 2 if in_width == 16 else 4

    def body(aggb_ref, disr_ref, w_ref, b_ref, g_ref, be_ref, out_ref):
        dis, y = _layer_y(aggb_ref, disr_ref, w_ref, b_ref, g_ref, be_ref,
                          in_width)
        t = dis * y
        for q in range(4):
            out_ref[q] = t[:, 16 * q:16 * q + 16]

    return pl.pallas_call(
        body,
        grid=(NP // RBLK,),
        in_specs=[
            pl.BlockSpec((nin, RBLK, 16), lambda i: (0, i, 0)),
            pl.BlockSpec((RBLK, 16), lambda i: (i, 0)),
            pl.BlockSpec(w.shape, lambda i: (0, 0)),
        ] + _VEC_SPECS,
        out_specs=pl.BlockSpec((4, RBLK, 16), lambda i: (0, i, 0)),
        out_shape=jax.ShapeDtypeStruct((4, NP, 16), jnp.float32),
    )(aggb, disr, w, b, g, be)


def _tc_layer_pool(aggb, disr, w, b, g, be, batch3, fc1w, fc1b, fc2v, fc2b):
    nblk = NP // RBLK

    def body(aggb_ref, disr_ref, w_ref, b_ref, g_ref, be_ref, bt_ref,
             w1_ref, b1_ref, w2_ref, b2_ref, out_ref, psum_ref, cnt_ref):
        _, y = _layer_y(aggb_ref, disr_ref, w_ref, b_ref, g_ref, be_ref, 64)
        bt = bt_ref[0]
        gi = lax.broadcasted_iota(jnp.int32, (G, RBLK), 0)
        one_t = (gi == bt).astype(jnp.float32)

        @pl.when(pl.program_id(0) == 0)
        def _():
            psum_ref[...] = jnp.zeros((G, 64), jnp.float32)
            cnt_ref[...] = jnp.zeros((G, 1), jnp.float32)

        psum_ref[...] += jnp.dot(one_t, y, preferred_element_type=jnp.float32)
        cnt_ref[...] += jnp.sum(one_t, axis=1, keepdims=True)

        @pl.when(pl.program_id(0) == nblk - 1)
        def _():
            pooled = psum_ref[...] / jnp.maximum(cnt_ref[...], 1.0)
            h = jnp.maximum(
                jnp.dot(pooled, w1_ref[...], preferred_element_type=jnp.float32)
                + b1_ref[...],
                0.0,
            )
            out_ref[...] = (
                jnp.sum(h * w2_ref[...], axis=1, keepdims=True) + b2_ref[...]
            )

    out, _, _ = pl.pallas_call(
        body,
        grid=(nblk,),
        in_specs=[
            pl.BlockSpec((4, RBLK, 16), lambda i: (0, i, 0)),
            pl.BlockSpec((RBLK, 16), lambda i: (i, 0)),
            pl.BlockSpec(w.shape, lambda i: (0, 0)),
        ] + _VEC_SPECS + [
            pl.BlockSpec((1, 1, RBLK), lambda i: (i, 0, 0)),
            pl.BlockSpec((64, 32), lambda i: (0, 0)),
            pl.BlockSpec((1, 32), lambda i: (0, 0)),
            pl.BlockSpec((1, 32), lambda i: (0, 0)),
            pl.BlockSpec((1, 1), lambda i: (0, 0)),
        ],
        out_specs=[
            pl.BlockSpec((G, 1), lambda i: (0, 0)),
            pl.BlockSpec((G, 64), lambda i: (0, 0)),
            pl.BlockSpec((G, 1), lambda i: (0, 0)),
        ],
        out_shape=[
            jax.ShapeDtypeStruct((G, 1), jnp.float32),
            jax.ShapeDtypeStruct((G, 64), jnp.float32),
            jax.ShapeDtypeStruct((G, 1), jnp.float32),
        ],
    )(aggb, disr, w, b, g, be, batch3, fc1w, fc1b, fc2v, fc2b)
    return out


def kernel(x, edge_index, batch, W0, b0, g0, be0, W1, b1, g1, be1,
           W2, b2, g2, be2, fc1_W, fc1_b, fc2_W, fc2_b):
    i32 = jnp.int32
    loop = jnp.arange(N, dtype=i32)
    src = jnp.concatenate([edge_index[0].astype(i32), loop])
    dst = jnp.concatenate([edge_index[1].astype(i32), loop])
    src_p = jnp.pad(src, (0, EP - ET), constant_values=N)
    dst_p = jnp.pad(dst, (0, EP - ET), constant_values=N)
    dst_ch = dst_p.reshape(NCH, CH)
    batch3 = jnp.pad(batch.astype(i32), (0, NP - N), constant_values=G).reshape(
        NP // RBLK, 1, RBLK
    )
    xp = jnp.zeros((NP, 16), jnp.float32).at[:N, :9].set(x)
    W0p = jnp.zeros((16, 64), jnp.float32).at[:9, :].set(W0)

    degb = _sc_deg(dst_ch)
    t0, disr = _tc_prep(degb, xp)
    agg0 = _sc_agg(t0, t0, src_p, dst_p, 16, feature_split=False, ub=8,
                   chw=128)
    t1q = _tc_layer(agg0, disr, W0p, b0.reshape(1, 64), g0.reshape(1, 64),
                    be0.reshape(1, 64), in_width=16)
    agg1 = _sc_agg_sp(t1q, src_p, dst_p)
    t2q = _tc_layer(agg1, disr, W1, b1.reshape(1, 64), g1.reshape(1, 64),
                    be1.reshape(1, 64), in_width=64)
    agg2 = _sc_agg_sp(t2q, src_p, dst_p)
    return _tc_layer_pool(agg2, disr, W2, b2.reshape(1, 64),
                          g2.reshape(1, 64), be2.reshape(1, 64), batch3,
                          fc1_W, fc1_b.reshape(1, 32), fc2_W.reshape(1, 32),
                          fc2_b.reshape(1, 1))


# speedup vs baseline: 1.3912x; 1.3912x over previous
import functools

import jax
import jax.numpy as jnp
import numpy as np
from jax import lax
from jax.experimental import pallas as pl
from jax.experimental.pallas import tpu as pltpu
from jax.experimental.pallas import tpu_sc as plsc

N = 50000
G = 512
EPS = 1e-5

NP = 51200
RPT = NP // 16
E_RAW = 800000
ET = E_RAW + N
CH = 128
UB = 8
EP = 851968
NCH = EP // CH
NODE_CH = NP // CH
NODE_SLABS = NODE_CH // UB
G2 = 1024
GPT = G2 // 16
RBLK = 512


def _mesh():
    return plsc.VectorSubcoreMesh(core_axis_name="c", subcore_axis_name="s")


_SC_PARAMS = pltpu.CompilerParams(use_tc_tiling_on_sc=False)


def _on_core(c, fn):
    @pl.when(c == 0)
    def _():
        fn(0)

    @pl.when(c == 1)
    def _():
        fn(1)


def _fill(ref, rows, width, val):
    @pl.loop(0, rows)
    def _(i):
        for h in range(width // 16):
            ref[i, pl.ds(h * 16, 16)] = jnp.full((16,), val, jnp.float32)


def _sc_deg(dst_ch):
    cpt = NCH // 32

    @functools.partial(
        pl.kernel,
        out_type=jax.ShapeDtypeStruct((2, NP, 16), jnp.float32),
        mesh=_mesh(),
        compiler_params=_SC_PARAMS,
        scratch_types=[
            pltpu.VMEM_SHARED((NP, 16), jnp.float32),
            pltpu.VMEM((UB, CH), jnp.int32),
            pltpu.VMEM((CH, 16), jnp.float32),
            pltpu.VMEM((CH, 16), jnp.float32),
        ],
    )
    def k(dst, deg_out, acc, idxb, ones, bounce):
        c = lax.axis_index("c")
        s = lax.axis_index("s")
        _fill(ones, CH, 16, 1.0)
        _fill(bounce, CH, 16, 0.0)
        @pl.loop(0, RPT // CH)
        def _(j):
            pltpu.sync_copy(bounce, acc.at[pl.ds(s * RPT + j * CH, CH)])

        plsc.subcore_barrier()

        tile_c0 = (c * 16 + s) * cpt

        @pl.loop(0, cpt // UB)
        def _(i):
            pltpu.sync_copy(dst.at[pl.ds(tile_c0 + i * UB, UB)], idxb)
            for q in range(UB):
                pltpu.sync_copy(ones, acc.at[idxb.at[q]], add=True)

        plsc.subcore_barrier()

        def wout(cc):
            @pl.loop(0, RPT // CH)
            def _(j):
                r = s * RPT + j * CH
                pltpu.sync_copy(acc.at[pl.ds(r, CH)], bounce)
                pltpu.sync_copy(bounce, deg_out.at[cc, pl.ds(r, CH)])

        _on_core(c, wout)

    return k(dst_ch)


def _sc_agg(tab0_arr, tab1_arr, src_p, dst_p, width, feature_split, ub, chw):
    nch = EP // chw
    if feature_split:
        cpt = nch // 16
    else:
        cpt = nch // 32
    nit = cpt // ub
    assert nit % 2 == 0
    zr = 64
    src_ch = src_p.reshape(nch, chw)
    dst_ch = dst_p.reshape(nch, chw)

    @functools.partial(
        pl.kernel,
        out_type=jax.ShapeDtypeStruct((2, NP, width), jnp.float32),
        mesh=_mesh(),
        compiler_params=_SC_PARAMS,
        scratch_types=[
            pltpu.VMEM_SHARED((NP, width), jnp.float32),
            pltpu.VMEM((2, ub, chw), jnp.int32),
            pltpu.VMEM((2, ub, chw), jnp.int32),
            pltpu.VMEM((2, ub, chw, width), jnp.float32),
            pltpu.VMEM((zr, width), jnp.float32),
            pltpu.SemaphoreType.DMA,
        ],
    )
    def k(tab0, tab1, idx, dst, out, acc, srcb, dstb, rows, zb, sem):
        c = lax.axis_index("c")
        s = lax.axis_index("s")
        _fill(zb, zr, width, 0.0)

        @pl.loop(0, RPT // zr)
        def _(j):
            pltpu.sync_copy(zb, acc.at[pl.ds(s * RPT + j * zr, zr)])

        plsc.subcore_barrier()

        if feature_split:
            tile_c0 = s * cpt
        else:
            tile_c0 = None

        def run(cc):
            c0 = tile_c0 if feature_split else (cc * 16 + s) * cpt
            tab = tab0 if cc == 0 else tab1

            def load(b, it):
                ch0 = c0 + it * ub
                pltpu.sync_copy(idx.at[pl.ds(ch0, ub)], srcb.at[b])
                pltpu.sync_copy(dst.at[pl.ds(ch0, ub)], dstb.at[b])

            def fire(b):
                for q in range(ub):
                    pltpu.async_copy(tab.at[srcb.at[b, q]], rows.at[b, q], sem)

            def drain(b):
                for q in range(ub):
                    pltpu.make_async_copy(
                        tab.at[srcb.at[b, q]], rows.at[b, q], sem
                    ).wait()

            def scat(b):
                for q in range(ub):
                    pltpu.sync_copy(rows.at[b, q], acc.at[dstb.at[b, q]], add=True)

            load(0, 0)
            fire(0)

            @pl.loop(0, nit // 2)
            def _(i2):
                i = i2 * 2
                drain(0)
                load(1, i + 1)
                fire(1)
                scat(0)
                drain(1)

                @pl.when(i + 2 < nit)
                def _():
                    load(0, i + 2)
                    fire(0)

                scat(1)


        _on_core(c, run)
        plsc.subcore_barrier()

        def wout(cc):
            @pl.loop(0, RPT // zr)
            def _(j):
                r = s * RPT + j * zr
                pltpu.sync_copy(acc.at[pl.ds(r, zr)], zb)
                pltpu.sync_copy(zb, out.at[cc, pl.ds(r, zr)])

        _on_core(c, wout)

    return k(tab0_arr, tab1_arr, src_ch, dst_ch)


def _tc_prep(degb, xp128):
    rows = NP * 16 // 128
    rb = rows // (NP // RBLK)

    def body(degb_ref, xp_ref, t0_ref, disr_ref):
        deg = degb_ref[0] + degb_ref[1]
        dis = jnp.where(deg > 0.0, lax.rsqrt(deg), 0.0)
        t0_ref[...] = dis * xp_ref[...]
        disr_ref[...] = dis

    degb128 = degb.reshape(2, rows, 128)
    t0, disr = pl.pallas_call(
        body,
        grid=(NP // RBLK,),
        in_specs=[
            pl.BlockSpec((2, rb, 128), lambda i: (0, i, 0)),
            pl.BlockSpec((rb, 128), lambda i: (i, 0)),
        ],
        out_specs=[
            pl.BlockSpec((rb, 128), lambda i: (i, 0)),
            pl.BlockSpec((rb, 128), lambda i: (i, 0)),
        ],
        out_shape=[jax.ShapeDtypeStruct((rows, 128), jnp.float32)] * 2,
    )(degb128, xp128)
    return t0.reshape(NP, 16), disr.reshape(NP, 16)


def _layer_y(aggb_ref, disr_ref, w_ref, b_ref, g_ref, be_ref, in_width):
    disr = disr_ref[...]
    dis4 = jnp.concatenate([disr, disr, disr, disr], axis=1)
    if in_width == 16:
        sx = disr * (aggb_ref[0] + aggb_ref[1])
    else:
        sx = dis4 * jnp.concatenate([aggb_ref[0], aggb_ref[1]], axis=1)
    z = jnp.dot(sx, w_ref[...], preferred_element_type=jnp.float32)
    alpha = g_ref[...] * np.float32(1.0 / np.sqrt(1.0 + EPS))
    y = jnp.maximum(alpha * (z + b_ref[...]) + be_ref[...], 0.0)
    return dis4, y


_VEC_SPECS = [
    pl.BlockSpec((1, 64), lambda i: (0, 0)),
    pl.BlockSpec((1, 64), lambda i: (0, 0)),
    pl.BlockSpec((1, 64), lambda i: (0, 0)),
]


def _tc_layer(aggb, disr, w, b, g, be, in_width):
    def body(aggb_ref, disr_ref, w_ref, b_ref, g_ref, be_ref, o0_ref, o1_ref):
        dis4, y = _layer_y(aggb_ref, disr_ref, w_ref, b_ref, g_ref, be_ref,
                           in_width)
        t = dis4 * y
        o0_ref[...] = t[:, :32]
        o1_ref[...] = t[:, 32:]

    return pl.pallas_call(
        body,
        grid=(NP // RBLK,),
        in_specs=[
            pl.BlockSpec((2, RBLK, in_width), lambda i: (0, i, 0)),
            pl.BlockSpec((RBLK, 16), lambda i: (i, 0)),
            pl.BlockSpec(w.shape, lambda i: (0, 0)),
        ] + _VEC_SPECS,
        out_specs=[
            pl.BlockSpec((RBLK, 32), lambda i: (i, 0)),
            pl.BlockSpec((RBLK, 32), lambda i: (i, 0)),
        ],
        out_shape=[jax.ShapeDtypeStruct((NP, 32), jnp.float32)] * 2,
    )(aggb, disr, w, b, g, be)


def _tc_layer_pool(aggb, disr, w, b, g, be, batch3, fc1w, fc1b, fc2v, fc2b):
    nblk = NP // RBLK

    def body(aggb_ref, disr_ref, w_ref, b_ref, g_ref, be_ref, bt_ref,
             w1_ref, b1_ref, w2_ref, b2_ref, out_ref, psum_ref, cnt_ref):
        _, y = _layer_y(aggb_ref, disr_ref, w_ref, b_ref, g_ref, be_ref, 32)
        bt = bt_ref[0]
        gi = lax.broadcasted_iota(jnp.int32, (G, RBLK), 0)
        one_t = (gi == bt).astype(jnp.float32)

        @pl.when(pl.program_id(0) == 0)
        def _():
            psum_ref[...] = jnp.zeros((G, 64), jnp.float32)
            cnt_ref[...] = jnp.zeros((G, 1), jnp.float32)

        psum_ref[...] += jnp.dot(one_t, y, preferred_element_type=jnp.float32)
        cnt_ref[...] += jnp.sum(one_t, axis=1, keepdims=True)

        @pl.when(pl.program_id(0) == nblk - 1)
        def _():
            pooled = psum_ref[...] / jnp.maximum(cnt_ref[...], 1.0)
            h = jnp.maximum(
                jnp.dot(pooled, w1_ref[...], preferred_element_type=jnp.float32)
                + b1_ref[...],
                0.0,
            )
            out_ref[...] = (
                jnp.sum(h * w2_ref[...], axis=1, keepdims=True) + b2_ref[...]
            )

    out, _, _ = pl.pallas_call(
        body,
        grid=(nblk,),
        in_specs=[
            pl.BlockSpec((2, RBLK, 32), lambda i: (0, i, 0)),
            pl.BlockSpec((RBLK, 16), lambda i: (i, 0)),
            pl.BlockSpec(w.shape, lambda i: (0, 0)),
        ] + _VEC_SPECS + [
            pl.BlockSpec((1, 1, RBLK), lambda i: (i, 0, 0)),
            pl.BlockSpec((64, 32), lambda i: (0, 0)),
            pl.BlockSpec((1, 32), lambda i: (0, 0)),
            pl.BlockSpec((1, 32), lambda i: (0, 0)),
            pl.BlockSpec((1, 1), lambda i: (0, 0)),
        ],
        out_specs=[
            pl.BlockSpec((G, 1), lambda i: (0, 0)),
            pl.BlockSpec((G, 64), lambda i: (0, 0)),
            pl.BlockSpec((G, 1), lambda i: (0, 0)),
        ],
        out_shape=[
            jax.ShapeDtypeStruct((G, 1), jnp.float32),
            jax.ShapeDtypeStruct((G, 64), jnp.float32),
            jax.ShapeDtypeStruct((G, 1), jnp.float32),
        ],
    )(aggb, disr, w, b, g, be, batch3, fc1w, fc1b, fc2v, fc2b)
    return out


def kernel(x, edge_index, batch, W0, b0, g0, be0, W1, b1, g1, be1,
           W2, b2, g2, be2, fc1_W, fc1_b, fc2_W, fc2_b):
    i32 = jnp.int32
    loop = jnp.arange(N, dtype=i32)
    src = jnp.concatenate([edge_index[0].astype(i32), loop])
    dst = jnp.concatenate([edge_index[1].astype(i32), loop])
    src_p = jnp.pad(src, (0, EP - ET), constant_values=N)
    dst_p = jnp.pad(dst, (0, EP - ET), constant_values=N)
    dst_ch = dst_p.reshape(NCH, CH)
    batch3 = jnp.pad(batch.astype(i32), (0, NP - N), constant_values=G).reshape(
        NP // RBLK, 1, RBLK
    )
    xp128 = jnp.zeros((NP, 16), jnp.float32).at[:N, :9].set(x).reshape(NP * 16 // 128, 128)
    W0p = jnp.zeros((16, 64), jnp.float32).at[:9, :].set(W0)

    degb = _sc_deg(dst_ch)
    t0, disr = _tc_prep(degb, xp128)
    agg0 = _sc_agg(t0, t0, src_p, dst_p, 16, feature_split=False, ub=8,
                   chw=128)
    t1a, t1b = _tc_layer(agg0, disr, W0p, b0.reshape(1, 64), g0.reshape(1, 64),
                         be0.reshape(1, 64), in_width=16)
    agg1 = _sc_agg(t1a, t1b, src_p, dst_p, 32, feature_split=True, ub=1,
                   chw=256)
    t2a, t2b = _tc_layer(agg1, disr, W1, b1.reshape(1, 64), g1.reshape(1, 64),
                         be1.reshape(1, 64), in_width=32)
    agg2 = _sc_agg(t2a, t2b, src_p, dst_p, 32, feature_split=True, ub=1,
                   chw=256)
    return _tc_layer_pool(agg2, disr, W2, b2.reshape(1, 64),
                          g2.reshape(1, 64), be2.reshape(1, 64), batch3,
                          fc1_W, fc1_b.reshape(1, 32), fc2_W.reshape(1, 32),
                          fc2_b.reshape(1, 1))


# speedup vs baseline: 1.4870x; 1.0688x over previous
import functools

import jax
import jax.numpy as jnp
import numpy as np
from jax import lax
from jax.experimental import pallas as pl
from jax.experimental.pallas import tpu as pltpu
from jax.experimental.pallas import tpu_sc as plsc

N = 50000
G = 512
EPS = 1e-5

NP = 51200
RPT = NP // 16
E_RAW = 800000
ET = E_RAW + N
CH = 128
UB = 8
EP = 851968
NCH = EP // CH
NODE_CH = NP // CH
NODE_SLABS = NODE_CH // UB
G2 = 1024
GPT = G2 // 16
RBLK = 1024


def _mesh():
    return plsc.VectorSubcoreMesh(core_axis_name="c", subcore_axis_name="s")


_SC_PARAMS = pltpu.CompilerParams(use_tc_tiling_on_sc=False)


def _on_core(c, fn):
    @pl.when(c == 0)
    def _():
        fn(0)

    @pl.when(c == 1)
    def _():
        fn(1)


def _fill(ref, rows, width, val):
    @pl.loop(0, rows)
    def _(i):
        for h in range(width // 16):
            ref[i, pl.ds(h * 16, 16)] = jnp.full((16,), val, jnp.float32)


def _sc_deg(dst_ch):
    cpt = NCH // 32

    @functools.partial(
        pl.kernel,
        out_type=jax.ShapeDtypeStruct((2, NP, 16), jnp.float32),
        mesh=_mesh(),
        compiler_params=_SC_PARAMS,
        scratch_types=[
            pltpu.VMEM_SHARED((NP, 16), jnp.float32),
            pltpu.VMEM((UB, CH), jnp.int32),
            pltpu.VMEM((CH, 16), jnp.float32),
            pltpu.VMEM((CH, 16), jnp.float32),
        ],
    )
    def k(dst, deg_out, acc, idxb, ones, bounce):
        c = lax.axis_index("c")
        s = lax.axis_index("s")
        _fill(ones, CH, 16, 1.0)
        _fill(bounce, CH, 16, 0.0)
        @pl.loop(0, RPT // CH)
        def _(j):
            pltpu.sync_copy(bounce, acc.at[pl.ds(s * RPT + j * CH, CH)])

        plsc.subcore_barrier()

        tile_c0 = (c * 16 + s) * cpt

        @pl.loop(0, cpt // UB)
        def _(i):
            pltpu.sync_copy(dst.at[pl.ds(tile_c0 + i * UB, UB)], idxb)
            for q in range(UB):
                pltpu.sync_copy(ones, acc.at[idxb.at[q]], add=True)

        plsc.subcore_barrier()

        def wout(cc):
            @pl.loop(0, RPT // CH)
            def _(j):
                r = s * RPT + j * CH
                pltpu.sync_copy(acc.at[pl.ds(r, CH)], bounce)
                pltpu.sync_copy(bounce, deg_out.at[cc, pl.ds(r, CH)])

        _on_core(c, wout)

    return k(dst_ch)


def _sc_agg(tab0_arr, tab1_arr, src_p, dst_p, width, feature_split, ub, chw):
    nch = EP // chw
    if feature_split:
        cpt = nch // 16
    else:
        cpt = nch // 32
    nit = cpt // ub
    assert nit % 2 == 0
    zr = 64
    src_ch = src_p.reshape(nch, chw)
    dst_ch = dst_p.reshape(nch, chw)

    @functools.partial(
        pl.kernel,
        out_type=jax.ShapeDtypeStruct((2, NP, width), jnp.float32),
        mesh=_mesh(),
        compiler_params=_SC_PARAMS,
        scratch_types=[
            pltpu.VMEM_SHARED((NP, width), jnp.float32),
            pltpu.VMEM((2, ub, chw), jnp.int32),
            pltpu.VMEM((2, ub, chw), jnp.int32),
            pltpu.VMEM((2, ub, chw, width), jnp.float32),
            pltpu.VMEM((zr, width), jnp.float32),
            pltpu.SemaphoreType.DMA,
        ],
    )
    def k(tab0, tab1, idx, dst, out, acc, srcb, dstb, rows, zb, sem):
        c = lax.axis_index("c")
        s = lax.axis_index("s")
        _fill(zb, zr, width, 0.0)

        @pl.loop(0, RPT // zr)
        def _(j):
            pltpu.sync_copy(zb, acc.at[pl.ds(s * RPT + j * zr, zr)])

        plsc.subcore_barrier()

        if feature_split:
            tile_c0 = s * cpt
        else:
            tile_c0 = None

        def run(cc):
            c0 = tile_c0 if feature_split else (cc * 16 + s) * cpt
            tab = tab0 if cc == 0 else tab1

            def load(b, it):
                ch0 = c0 + it * ub
                pltpu.sync_copy(idx.at[pl.ds(ch0, ub)], srcb.at[b])
                pltpu.sync_copy(dst.at[pl.ds(ch0, ub)], dstb.at[b])

            def fire(b):
                for q in range(ub):
                    pltpu.async_copy(tab.at[srcb.at[b, q]], rows.at[b, q], sem)

            def drain(b):
                for q in range(ub):
                    pltpu.make_async_copy(
                        tab.at[srcb.at[b, q]], rows.at[b, q], sem
                    ).wait()

            def scat(b):
                for q in range(ub):
                    pltpu.sync_copy(rows.at[b, q], acc.at[dstb.at[b, q]], add=True)

            load(0, 0)
            fire(0)

            @pl.loop(0, nit // 2)
            def _(i2):
                i = i2 * 2
                drain(0)
                load(1, i + 1)
                fire(1)
                scat(0)
                drain(1)

                @pl.when(i + 2 < nit)
                def _():
                    load(0, i + 2)
                    fire(0)

                scat(1)


        _on_core(c, run)
        plsc.subcore_barrier()

        def wout(cc):
            @pl.loop(0, RPT // zr)
            def _(j):
                r = s * RPT + j * zr
                pltpu.sync_copy(acc.at[pl.ds(r, zr)], zb)
                pltpu.sync_copy(zb, out.at[cc, pl.ds(r, zr)])

        _on_core(c, wout)

    return k(tab0_arr, tab1_arr, src_ch, dst_ch)


def _tc_prep(degb, xp128):
    rows = NP * 16 // 128
    rb = rows // (NP // RBLK)

    def body(degb_ref, xp_ref, t0_ref, disr_ref):
        deg = degb_ref[0] + degb_ref[1]
        dis = jnp.where(deg > 0.0, lax.rsqrt(deg), 0.0)
        t0_ref[...] = dis * xp_ref[...]
        disr_ref[...] = dis

    degb128 = degb.reshape(2, rows, 128)
    t0, disr = pl.pallas_call(
        body,
        grid=(NP // RBLK,),
        in_specs=[
            pl.BlockSpec((2, rb, 128), lambda i: (0, i, 0)),
            pl.BlockSpec((rb, 128), lambda i: (i, 0)),
        ],
        out_specs=[
            pl.BlockSpec((rb, 128), lambda i: (i, 0)),
            pl.BlockSpec((rb, 128), lambda i: (i, 0)),
        ],
        out_shape=[jax.ShapeDtypeStruct((rows, 128), jnp.float32)] * 2,
    )(degb128, xp128)
    return t0.reshape(NP, 16), disr.reshape(NP, 16)


def _layer_y(aggb_ref, disr_ref, w_ref, b_ref, g_ref, be_ref, in_width):
    disr = disr_ref[...]
    dis4 = jnp.concatenate([disr, disr, disr, disr], axis=1)
    if in_width == 16:
        sx = disr * (aggb_ref[0] + aggb_ref[1])
    else:
        sx = dis4 * jnp.concatenate([aggb_ref[0], aggb_ref[1]], axis=1)
    z = jnp.dot(sx, w_ref[...], preferred_element_type=jnp.float32)
    alpha = g_ref[...] * np.float32(1.0 / np.sqrt(1.0 + EPS))
    y = jnp.maximum(alpha * (z + b_ref[...]) + be_ref[...], 0.0)
    return dis4, y


_VEC_SPECS = [
    pl.BlockSpec((1, 64), lambda i: (0, 0)),
    pl.BlockSpec((1, 64), lambda i: (0, 0)),
    pl.BlockSpec((1, 64), lambda i: (0, 0)),
]


def _tc_layer(aggb, disr, w, b, g, be, in_width):
    def body(aggb_ref, disr_ref, w_ref, b_ref, g_ref, be_ref, o0_ref, o1_ref):
        dis4, y = _layer_y(aggb_ref, disr_ref, w_ref, b_ref, g_ref, be_ref,
                           in_width)
        t = dis4 * y
        o0_ref[...] = t[:, :32]
        o1_ref[...] = t[:, 32:]

    return pl.pallas_call(
        body,
        grid=(NP // RBLK,),
        in_specs=[
            pl.BlockSpec((2, RBLK, in_width), lambda i: (0, i, 0)),
            pl.BlockSpec((RBLK, 16), lambda i: (i, 0)),
            pl.BlockSpec(w.shape, lambda i: (0, 0)),
        ] + _VEC_SPECS,
        out_specs=[
            pl.BlockSpec((RBLK, 32), lambda i: (i, 0)),
            pl.BlockSpec((RBLK, 32), lambda i: (i, 0)),
        ],
        out_shape=[jax.ShapeDtypeStruct((NP, 32), jnp.float32)] * 2,
    )(aggb, disr, w, b, g, be)


def _tc_layer_pool(aggb, disr, w, b, g, be, batch3, fc1w, fc1b, fc2v, fc2b):
    nblk = NP // RBLK

    def body(aggb_ref, disr_ref, w_ref, b_ref, g_ref, be_ref, bt_ref,
             w1_ref, b1_ref, w2_ref, b2_ref, out_ref, psum_ref, cnt_ref):
        _, y = _layer_y(aggb_ref, disr_ref, w_ref, b_ref, g_ref, be_ref, 32)
        bt = bt_ref[0]
        gi = lax.broadcasted_iota(jnp.int32, (G, RBLK), 0)
        one_t = (gi == bt).astype(jnp.float32)

        @pl.when(pl.program_id(0) == 0)
        def _():
            psum_ref[...] = jnp.zeros((G, 64), jnp.float32)
            cnt_ref[...] = jnp.zeros((G, 1), jnp.float32)

        psum_ref[...] += jnp.dot(one_t, y, preferred_element_type=jnp.float32)
        cnt_ref[...] += jnp.sum(one_t, axis=1, keepdims=True)

        @pl.when(pl.program_id(0) == nblk - 1)
        def _():
            pooled = psum_ref[...] / jnp.maximum(cnt_ref[...], 1.0)
            h = jnp.maximum(
                jnp.dot(pooled, w1_ref[...], preferred_element_type=jnp.float32)
                + b1_ref[...],
                0.0,
            )
            out_ref[...] = (
                jnp.sum(h * w2_ref[...], axis=1, keepdims=True) + b2_ref[...]
            )

    out, _, _ = pl.pallas_call(
        body,
        grid=(nblk,),
        in_specs=[
            pl.BlockSpec((2, RBLK, 32), lambda i: (0, i, 0)),
            pl.BlockSpec((RBLK, 16), lambda i: (i, 0)),
            pl.BlockSpec(w.shape, lambda i: (0, 0)),
        ] + _VEC_SPECS + [
            pl.BlockSpec((1, 1, RBLK), lambda i: (i, 0, 0)),
            pl.BlockSpec((64, 32), lambda i: (0, 0)),
            pl.BlockSpec((1, 32), lambda i: (0, 0)),
            pl.BlockSpec((1, 32), lambda i: (0, 0)),
            pl.BlockSpec((1, 1), lambda i: (0, 0)),
        ],
        out_specs=[
            pl.BlockSpec((G, 1), lambda i: (0, 0)),
            pl.BlockSpec((G, 64), lambda i: (0, 0)),
            pl.BlockSpec((G, 1), lambda i: (0, 0)),
        ],
        out_shape=[
            jax.ShapeDtypeStruct((G, 1), jnp.float32),
            jax.ShapeDtypeStruct((G, 64), jnp.float32),
            jax.ShapeDtypeStruct((G, 1), jnp.float32),
        ],
    )(aggb, disr, w, b, g, be, batch3, fc1w, fc1b, fc2v, fc2b)
    return out


def kernel(x, edge_index, batch, W0, b0, g0, be0, W1, b1, g1, be1,
           W2, b2, g2, be2, fc1_W, fc1_b, fc2_W, fc2_b):
    i32 = jnp.int32
    loop = jnp.arange(N, dtype=i32)
    src = jnp.concatenate([edge_index[0].astype(i32), loop])
    dst = jnp.concatenate([edge_index[1].astype(i32), loop])
    src_p = jnp.pad(src, (0, EP - ET), constant_values=N)
    dst_p = jnp.pad(dst, (0, EP - ET), constant_values=N)
    dst_ch = dst_p.reshape(NCH, CH)
    batch3 = jnp.pad(batch.astype(i32), (0, NP - N), constant_values=G).reshape(
        NP // RBLK, 1, RBLK
    )
    xp128 = jnp.zeros((NP, 16), jnp.float32).at[:N, :9].set(x).reshape(NP * 16 // 128, 128)
    W0p = jnp.zeros((16, 64), jnp.float32).at[:9, :].set(W0)

    degb = _sc_deg(dst_ch)
    t0, disr = _tc_prep(degb, xp128)
    agg0 = _sc_agg(t0, t0, src_p, dst_p, 16, feature_split=False, ub=8,
                   chw=128)
    t1a, t1b = _tc_layer(agg0, disr, W0p, b0.reshape(1, 64), g0.reshape(1, 64),
                         be0.reshape(1, 64), in_width=16)
    agg1 = _sc_agg(t1a, t1b, src_p, dst_p, 32, feature_split=True, ub=1,
                   chw=256)
    t2a, t2b = _tc_layer(agg1, disr, W1, b1.reshape(1, 64), g1.reshape(1, 64),
                         be1.reshape(1, 64), in_width=32)
    agg2 = _sc_agg(t2a, t2b, src_p, dst_p, 32, feature_split=True, ub=1,
                   chw=256)
    return _tc_layer_pool(agg2, disr, W2, b2.reshape(1, 64),
                          g2.reshape(1, 64), be2.reshape(1, 64), batch3,
                          fc1_W, fc1_b.reshape(1, 32), fc2_W.reshape(1, 32),
                          fc2_b.reshape(1, 1))


# speedup vs baseline: 1.5380x; 1.0344x over previous
import functools

import jax
import jax.numpy as jnp
import numpy as np
from jax import lax
from jax.experimental import pallas as pl
from jax.experimental.pallas import tpu as pltpu
from jax.experimental.pallas import tpu_sc as plsc

N = 50000
G = 512
EPS = 1e-5

NP = 51200
RPT = NP // 16
E_RAW = 800000
ET = E_RAW + N
CH = 128
UB = 8
EP = 851968
NCH = EP // CH
NODE_CH = NP // CH
NODE_SLABS = NODE_CH // UB
G2 = 1024
GPT = G2 // 16
RBLK = 2048


def _mesh():
    return plsc.VectorSubcoreMesh(core_axis_name="c", subcore_axis_name="s")


_SC_PARAMS = pltpu.CompilerParams(use_tc_tiling_on_sc=False)


def _on_core(c, fn):
    @pl.when(c == 0)
    def _():
        fn(0)

    @pl.when(c == 1)
    def _():
        fn(1)


def _fill(ref, rows, width, val):
    @pl.loop(0, rows)
    def _(i):
        for h in range(width // 16):
            ref[i, pl.ds(h * 16, 16)] = jnp.full((16,), val, jnp.float32)


def _sc_deg(dst_ch):
    cpt = NCH // 32

    @functools.partial(
        pl.kernel,
        out_type=jax.ShapeDtypeStruct((2, NP, 16), jnp.float32),
        mesh=_mesh(),
        compiler_params=_SC_PARAMS,
        scratch_types=[
            pltpu.VMEM_SHARED((NP, 16), jnp.float32),
            pltpu.VMEM((UB, CH), jnp.int32),
            pltpu.VMEM((CH, 16), jnp.float32),
            pltpu.VMEM((CH, 16), jnp.float32),
        ],
    )
    def k(dst, deg_out, acc, idxb, ones, bounce):
        c = lax.axis_index("c")
        s = lax.axis_index("s")
        _fill(ones, CH, 16, 1.0)
        _fill(bounce, CH, 16, 0.0)
        @pl.loop(0, RPT // CH)
        def _(j):
            pltpu.sync_copy(bounce, acc.at[pl.ds(s * RPT + j * CH, CH)])

        plsc.subcore_barrier()

        tile_c0 = (c * 16 + s) * cpt

        @pl.loop(0, cpt // UB)
        def _(i):
            pltpu.sync_copy(dst.at[pl.ds(tile_c0 + i * UB, UB)], idxb)
            for q in range(UB):
                pltpu.sync_copy(ones, acc.at[idxb.at[q]], add=True)

        plsc.subcore_barrier()

        def wout(cc):
            @pl.loop(0, RPT // CH)
            def _(j):
                r = s * RPT + j * CH
                pltpu.sync_copy(acc.at[pl.ds(r, CH)], bounce)
                pltpu.sync_copy(bounce, deg_out.at[cc, pl.ds(r, CH)])

        _on_core(c, wout)

    return k(dst_ch)


def _sc_agg(tab0_arr, tab1_arr, src_p, dst_p, width, feature_split, ub, chw):
    nch = EP // chw
    if feature_split:
        cpt = nch // 16
    else:
        cpt = nch // 32
    nit = cpt // ub
    assert nit % 2 == 0
    zr = 64
    src_ch = src_p.reshape(nch, chw)
    dst_ch = dst_p.reshape(nch, chw)

    @functools.partial(
        pl.kernel,
        out_type=jax.ShapeDtypeStruct((2, NP, width), jnp.float32),
        mesh=_mesh(),
        compiler_params=_SC_PARAMS,
        scratch_types=[
            pltpu.VMEM_SHARED((NP, width), jnp.float32),
            pltpu.VMEM((2, ub, chw), jnp.int32),
            pltpu.VMEM((2, ub, chw), jnp.int32),
            pltpu.VMEM((2, ub, chw, width), jnp.float32),
            pltpu.VMEM((zr, width), jnp.float32),
            pltpu.SemaphoreType.DMA,
        ],
    )
    def k(tab0, tab1, idx, dst, out, acc, srcb, dstb, rows, zb, sem):
        c = lax.axis_index("c")
        s = lax.axis_index("s")
        _fill(zb, zr, width, 0.0)

        @pl.loop(0, RPT // zr)
        def _(j):
            pltpu.sync_copy(zb, acc.at[pl.ds(s * RPT + j * zr, zr)])

        plsc.subcore_barrier()

        if feature_split:
            tile_c0 = s * cpt
        else:
            tile_c0 = None

        def run(cc):
            c0 = tile_c0 if feature_split else (cc * 16 + s) * cpt
            tab = tab0 if cc == 0 else tab1

            def load(b, it):
                ch0 = c0 + it * ub
                pltpu.sync_copy(idx.at[pl.ds(ch0, ub)], srcb.at[b])
                pltpu.sync_copy(dst.at[pl.ds(ch0, ub)], dstb.at[b])

            def fire(b):
                for q in range(ub):
                    pltpu.async_copy(tab.at[srcb.at[b, q]], rows.at[b, q], sem)

            def drain(b):
                for q in range(ub):
                    pltpu.make_async_copy(
                        tab.at[srcb.at[b, q]], rows.at[b, q], sem
                    ).wait()

            def scat(b):
                for q in range(ub):
                    pltpu.sync_copy(rows.at[b, q], acc.at[dstb.at[b, q]], add=True)

            load(0, 0)
            fire(0)

            @pl.loop(0, nit // 2)
            def _(i2):
                i = i2 * 2
                drain(0)
                load(1, i + 1)
                fire(1)
                scat(0)
                drain(1)

                @pl.when(i + 2 < nit)
                def _():
                    load(0, i + 2)
                    fire(0)

                scat(1)


        _on_core(c, run)
        plsc.subcore_barrier()

        def wout(cc):
            @pl.loop(0, RPT // zr)
            def _(j):
                r = s * RPT + j * zr
                pltpu.sync_copy(acc.at[pl.ds(r, zr)], zb)
                pltpu.sync_copy(zb, out.at[cc, pl.ds(r, zr)])

        _on_core(c, wout)

    return k(tab0_arr, tab1_arr, src_ch, dst_ch)


def _tc_prep(degb, xp128):
    rows = NP * 16 // 128
    rb = rows // (NP // RBLK)

    def body(degb_ref, xp_ref, t0_ref, disr_ref):
        deg = degb_ref[0] + degb_ref[1]
        dis = jnp.where(deg > 0.0, lax.rsqrt(deg), 0.0)
        t0_ref[...] = dis * xp_ref[...]
        disr_ref[...] = dis

    degb128 = degb.reshape(2, rows, 128)
    t0, disr = pl.pallas_call(
        body,
        grid=(NP // RBLK,),
        in_specs=[
            pl.BlockSpec((2, rb, 128), lambda i: (0, i, 0)),
            pl.BlockSpec((rb, 128), lambda i: (i, 0)),
        ],
        out_specs=[
            pl.BlockSpec((rb, 128), lambda i: (i, 0)),
            pl.BlockSpec((rb, 128), lambda i: (i, 0)),
        ],
        out_shape=[jax.ShapeDtypeStruct((rows, 128), jnp.float32)] * 2,
    )(degb128, xp128)
    return t0.reshape(NP, 16), disr.reshape(NP, 16)


def _layer_y(aggb_ref, disr_ref, w_ref, b_ref, g_ref, be_ref, in_width):
    disr = disr_ref[...]
    dis4 = jnp.concatenate([disr, disr, disr, disr], axis=1)
    if in_width == 16:
        sx = disr * (aggb_ref[0] + aggb_ref[1])
    else:
        sx = dis4 * jnp.concatenate([aggb_ref[0], aggb_ref[1]], axis=1)
    z = jnp.dot(sx, w_ref[...], preferred_element_type=jnp.float32)
    alpha = g_ref[...] * np.float32(1.0 / np.sqrt(1.0 + EPS))
    y = jnp.maximum(alpha * (z + b_ref[...]) + be_ref[...], 0.0)
    return dis4, y


_VEC_SPECS = [
    pl.BlockSpec((1, 64), lambda i: (0, 0)),
    pl.BlockSpec((1, 64), lambda i: (0, 0)),
    pl.BlockSpec((1, 64), lambda i: (0, 0)),
]


def _tc_layer(aggb, disr, w, b, g, be, in_width):
    def body(aggb_ref, disr_ref, w_ref, b_ref, g_ref, be_ref, o0_ref, o1_ref):
        dis4, y = _layer_y(aggb_ref, disr_ref, w_ref, b_ref, g_ref, be_ref,
                           in_width)
        t = dis4 * y
        o0_ref[...] = t[:, :32]
        o1_ref[...] = t[:, 32:]

    return pl.pallas_call(
        body,
        grid=(NP // RBLK,),
        in_specs=[
            pl.BlockSpec((2, RBLK, in_width), lambda i: (0, i, 0)),
            pl.BlockSpec((RBLK, 16), lambda i: (i, 0)),
            pl.BlockSpec(w.shape, lambda i: (0, 0)),
        ] + _VEC_SPECS,
        out_specs=[
            pl.BlockSpec((RBLK, 32), lambda i: (i, 0)),
            pl.BlockSpec((RBLK, 32), lambda i: (i, 0)),
        ],
        out_shape=[jax.ShapeDtypeStruct((NP, 32), jnp.float32)] * 2,
    )(aggb, disr, w, b, g, be)


def _tc_layer_pool(aggb, disr, w, b, g, be, batch3, fc1w, fc1b, fc2v, fc2b):
    nblk = NP // RBLK

    def body(aggb_ref, disr_ref, w_ref, b_ref, g_ref, be_ref, bt_ref,
             w1_ref, b1_ref, w2_ref, b2_ref, out_ref, psum_ref, cnt_ref):
        _, y = _layer_y(aggb_ref, disr_ref, w_ref, b_ref, g_ref, be_ref, 32)
        bt = bt_ref[0]
        gi = lax.broadcasted_iota(jnp.int32, (G, RBLK), 0)
        one_t = (gi == bt).astype(jnp.float32)

        @pl.when(pl.program_id(0) == 0)
        def _():
            psum_ref[...] = jnp.zeros((G, 64), jnp.float32)
            cnt_ref[...] = jnp.zeros((G, 1), jnp.float32)

        psum_ref[...] += jnp.dot(one_t, y, preferred_element_type=jnp.float32)
        cnt_ref[...] += jnp.sum(one_t, axis=1, keepdims=True)

        @pl.when(pl.program_id(0) == nblk - 1)
        def _():
            pooled = psum_ref[...] / jnp.maximum(cnt_ref[...], 1.0)
            h = jnp.maximum(
                jnp.dot(pooled, w1_ref[...], preferred_element_type=jnp.float32)
                + b1_ref[...],
                0.0,
            )
            out_ref[...] = (
                jnp.sum(h * w2_ref[...], axis=1, keepdims=True) + b2_ref[...]
            )

    out, _, _ = pl.pallas_call(
        body,
        grid=(nblk,),
        in_specs=[
            pl.BlockSpec((2, RBLK, 32), lambda i: (0, i, 0)),
            pl.BlockSpec((RBLK, 16), lambda i: (i, 0)),
            pl.BlockSpec(w.shape, lambda i: (0, 0)),
        ] + _VEC_SPECS + [
            pl.BlockSpec((1, 1, RBLK), lambda i: (i, 0, 0)),
            pl.BlockSpec((64, 32), lambda i: (0, 0)),
            pl.BlockSpec((1, 32), lambda i: (0, 0)),
            pl.BlockSpec((1, 32), lambda i: (0, 0)),
            pl.BlockSpec((1, 1), lambda i: (0, 0)),
        ],
        out_specs=[
            pl.BlockSpec((G, 1), lambda i: (0, 0)),
            pl.BlockSpec((G, 64), lambda i: (0, 0)),
            pl.BlockSpec((G, 1), lambda i: (0, 0)),
        ],
        out_shape=[
            jax.ShapeDtypeStruct((G, 1), jnp.float32),
            jax.ShapeDtypeStruct((G, 64), jnp.float32),
            jax.ShapeDtypeStruct((G, 1), jnp.float32),
        ],
    )(aggb, disr, w, b, g, be, batch3, fc1w, fc1b, fc2v, fc2b)
    return out


def kernel(x, edge_index, batch, W0, b0, g0, be0, W1, b1, g1, be1,
           W2, b2, g2, be2, fc1_W, fc1_b, fc2_W, fc2_b):
    i32 = jnp.int32
    loop = jnp.arange(N, dtype=i32)
    src = jnp.concatenate([edge_index[0].astype(i32), loop])
    dst = jnp.concatenate([edge_index[1].astype(i32), loop])
    src_p = jnp.pad(src, (0, EP - ET), constant_values=N)
    dst_p = jnp.pad(dst, (0, EP - ET), constant_values=N)
    dst_ch = dst_p.reshape(NCH, CH)
    batch3 = jnp.pad(batch.astype(i32), (0, NP - N), constant_values=G).reshape(
        NP // RBLK, 1, RBLK
    )
    xp128 = jnp.zeros((NP, 16), jnp.float32).at[:N, :9].set(x).reshape(NP * 16 // 128, 128)
    W0p = jnp.zeros((16, 64), jnp.float32).at[:9, :].set(W0)

    degb = _sc_deg(dst_ch)
    t0, disr = _tc_prep(degb, xp128)
    agg0 = _sc_agg(t0, t0, src_p, dst_p, 16, feature_split=False, ub=8,
                   chw=128)
    t1a, t1b = _tc_layer(agg0, disr, W0p, b0.reshape(1, 64), g0.reshape(1, 64),
                         be0.reshape(1, 64), in_width=16)
    agg1 = _sc_agg(t1a, t1b, src_p, dst_p, 32, feature_split=True, ub=1,
                   chw=256)
    t2a, t2b = _tc_layer(agg1, disr, W1, b1.reshape(1, 64), g1.reshape(1, 64),
                         be1.reshape(1, 64), in_width=32)
    agg2 = _sc_agg(t2a, t2b, src_p, dst_p, 32, feature_split=True, ub=1,
                   chw=256)
    return _tc_layer_pool(agg2, disr, W2, b2.reshape(1, 64),
                          g2.reshape(1, 64), be2.reshape(1, 64), batch3,
                          fc1_W, fc1_b.reshape(1, 32), fc2_W.reshape(1, 32),
                          fc2_b.reshape(1, 1))


# speedup vs baseline: 1.5542x; 1.0105x over previous
import functools

import jax
import jax.numpy as jnp
import numpy as np
from jax import lax
from jax.experimental import pallas as pl
from jax.experimental.pallas import tpu as pltpu
from jax.experimental.pallas import tpu_sc as plsc

N = 50000
G = 512
EPS = 1e-5

NP = 51200
RPT = NP // 16
E_RAW = 800000
ET = E_RAW + N
CH = 128
UB = 8
EP = 851968
NCH = EP // CH
NODE_CH = NP // CH
NODE_SLABS = NODE_CH // UB
G2 = 1024
GPT = G2 // 16
RBLK = 3200


def _mesh():
    return plsc.VectorSubcoreMesh(core_axis_name="c", subcore_axis_name="s")


_SC_PARAMS = pltpu.CompilerParams(use_tc_tiling_on_sc=False)


def _on_core(c, fn):
    @pl.when(c == 0)
    def _():
        fn(0)

    @pl.when(c == 1)
    def _():
        fn(1)


def _fill(ref, rows, width, val):
    @pl.loop(0, rows)
    def _(i):
        for h in range(width // 16):
            ref[i, pl.ds(h * 16, 16)] = jnp.full((16,), val, jnp.float32)


def _sc_deg(dst_ch):
    cpt = NCH // 32

    @functools.partial(
        pl.kernel,
        out_type=jax.ShapeDtypeStruct((2, NP, 16), jnp.float32),
        mesh=_mesh(),
        compiler_params=_SC_PARAMS,
        scratch_types=[
            pltpu.VMEM_SHARED((NP, 16), jnp.float32),
            pltpu.VMEM((UB, CH), jnp.int32),
            pltpu.VMEM((CH, 16), jnp.float32),
            pltpu.VMEM((CH, 16), jnp.float32),
        ],
    )
    def k(dst, deg_out, acc, idxb, ones, bounce):
        c = lax.axis_index("c")
        s = lax.axis_index("s")
        _fill(ones, CH, 16, 1.0)
        _fill(bounce, CH, 16, 0.0)
        @pl.loop(0, RPT // CH)
        def _(j):
            pltpu.sync_copy(bounce, acc.at[pl.ds(s * RPT + j * CH, CH)])

        plsc.subcore_barrier()

        tile_c0 = (c * 16 + s) * cpt

        @pl.loop(0, cpt // UB)
        def _(i):
            pltpu.sync_copy(dst.at[pl.ds(tile_c0 + i * UB, UB)], idxb)
            for q in range(UB):
                pltpu.sync_copy(ones, acc.at[idxb.at[q]], add=True)

        plsc.subcore_barrier()

        def wout(cc):
            @pl.loop(0, RPT // CH)
            def _(j):
                r = s * RPT + j * CH
                pltpu.sync_copy(acc.at[pl.ds(r, CH)], bounce)
                pltpu.sync_copy(bounce, deg_out.at[cc, pl.ds(r, CH)])

        _on_core(c, wout)

    return k(dst_ch)


def _sc_agg(tab0_arr, tab1_arr, src_p, dst_p, width, feature_split, ub, chw):
    nch = EP // chw
    if feature_split:
        cpt = nch // 16
    else:
        cpt = nch // 32
    nit = cpt // ub
    assert nit % 2 == 0
    zr = 64
    src_ch = src_p.reshape(nch, chw)
    dst_ch = dst_p.reshape(nch, chw)

    @functools.partial(
        pl.kernel,
        out_type=jax.ShapeDtypeStruct((2, NP, width), jnp.float32),
        mesh=_mesh(),
        compiler_params=_SC_PARAMS,
        scratch_types=[
            pltpu.VMEM_SHARED((NP, width), jnp.float32),
            pltpu.VMEM((2, ub, chw), jnp.int32),
            pltpu.VMEM((2, ub, chw), jnp.int32),
            pltpu.VMEM((2, ub, chw, width), jnp.float32),
            pltpu.VMEM((zr, width), jnp.float32),
            pltpu.SemaphoreType.DMA,
        ],
    )
    def k(tab0, tab1, idx, dst, out, acc, srcb, dstb, rows, zb, sem):
        c = lax.axis_index("c")
        s = lax.axis_index("s")
        _fill(zb, zr, width, 0.0)

        @pl.loop(0, RPT // zr)
        def _(j):
            pltpu.sync_copy(zb, acc.at[pl.ds(s * RPT + j * zr, zr)])

        plsc.subcore_barrier()

        if feature_split:
            tile_c0 = s * cpt
        else:
            tile_c0 = None

        def run(cc):
            c0 = tile_c0 if feature_split else (cc * 16 + s) * cpt
            tab = tab0 if cc == 0 else tab1

            def load(b, it):
                ch0 = c0 + it * ub
                pltpu.sync_copy(idx.at[pl.ds(ch0, ub)], srcb.at[b])
                pltpu.sync_copy(dst.at[pl.ds(ch0, ub)], dstb.at[b])

            def fire(b):
                for q in range(ub):
                    pltpu.async_copy(tab.at[srcb.at[b, q]], rows.at[b, q], sem)

            def drain(b):
                for q in range(ub):
                    pltpu.make_async_copy(
                        tab.at[srcb.at[b, q]], rows.at[b, q], sem
                    ).wait()

            def scat(b):
                for q in range(ub):
                    pltpu.sync_copy(rows.at[b, q], acc.at[dstb.at[b, q]], add=True)

            load(0, 0)
            fire(0)

            @pl.loop(0, nit // 2)
            def _(i2):
                i = i2 * 2
                drain(0)
                load(1, i + 1)
                fire(1)
                scat(0)
                drain(1)

                @pl.when(i + 2 < nit)
                def _():
                    load(0, i + 2)
                    fire(0)

                scat(1)


        _on_core(c, run)
        plsc.subcore_barrier()

        def wout(cc):
            @pl.loop(0, RPT // zr)
            def _(j):
                r = s * RPT + j * zr
                pltpu.sync_copy(acc.at[pl.ds(r, zr)], zb)
                pltpu.sync_copy(zb, out.at[cc, pl.ds(r, zr)])

        _on_core(c, wout)

    return k(tab0_arr, tab1_arr, src_ch, dst_ch)


def _tc_prep(degb, xp128):
    rows = NP * 16 // 128
    rb = rows // (NP // RBLK)

    def body(degb_ref, xp_ref, t0_ref, disr_ref):
        deg = degb_ref[0] + degb_ref[1]
        dis = jnp.where(deg > 0.0, lax.rsqrt(deg), 0.0)
        t0_ref[...] = dis * xp_ref[...]
        disr_ref[...] = dis

    degb128 = degb.reshape(2, rows, 128)
    t0, disr = pl.pallas_call(
        body,
        grid=(NP // RBLK,),
        in_specs=[
            pl.BlockSpec((2, rb, 128), lambda i: (0, i, 0)),
            pl.BlockSpec((rb, 128), lambda i: (i, 0)),
        ],
        out_specs=[
            pl.BlockSpec((rb, 128), lambda i: (i, 0)),
            pl.BlockSpec((rb, 128), lambda i: (i, 0)),
        ],
        out_shape=[jax.ShapeDtypeStruct((rows, 128), jnp.float32)] * 2,
    )(degb128, xp128)
    return t0.reshape(NP, 16), disr.reshape(NP, 16)


def _layer_y(aggb_ref, disr_ref, w_ref, b_ref, g_ref, be_ref, in_width):
    disr = disr_ref[...]
    dis4 = jnp.concatenate([disr, disr, disr, disr], axis=1)
    if in_width == 16:
        sx = disr * (aggb_ref[0] + aggb_ref[1])
    else:
        sx = dis4 * jnp.concatenate([aggb_ref[0], aggb_ref[1]], axis=1)
    z = jnp.dot(sx, w_ref[...], preferred_element_type=jnp.float32)
    alpha = g_ref[...] * np.float32(1.0 / np.sqrt(1.0 + EPS))
    y = jnp.maximum(alpha * (z + b_ref[...]) + be_ref[...], 0.0)
    return dis4, y


_VEC_SPECS = [
    pl.BlockSpec((1, 64), lambda i: (0, 0)),
    pl.BlockSpec((1, 64), lambda i: (0, 0)),
    pl.BlockSpec((1, 64), lambda i: (0, 0)),
]


def _tc_layer(aggb, disr, w, b, g, be, in_width):
    def body(aggb_ref, disr_ref, w_ref, b_ref, g_ref, be_ref, o0_ref, o1_ref):
        dis4, y = _layer_y(aggb_ref, disr_ref, w_ref, b_ref, g_ref, be_ref,
                           in_width)
        t = dis4 * y
        o0_ref[...] = t[:, :32]
        o1_ref[...] = t[:, 32:]

    return pl.pallas_call(
        body,
        grid=(NP // RBLK,),
        in_specs=[
            pl.BlockSpec((2, RBLK, in_width), lambda i: (0, i, 0)),
            pl.BlockSpec((RBLK, 16), lambda i: (i, 0)),
            pl.BlockSpec(w.shape, lambda i: (0, 0)),
        ] + _VEC_SPECS,
        out_specs=[
            pl.BlockSpec((RBLK, 32), lambda i: (i, 0)),
            pl.BlockSpec((RBLK, 32), lambda i: (i, 0)),
        ],
        out_shape=[jax.ShapeDtypeStruct((NP, 32), jnp.float32)] * 2,
    )(aggb, disr, w, b, g, be)


def _tc_layer_pool(aggb, disr, w, b, g, be, batch3, fc1w, fc1b, fc2v, fc2b):
    nblk = NP // RBLK

    def body(aggb_ref, disr_ref, w_ref, b_ref, g_ref, be_ref, bt_ref,
             w1_ref, b1_ref, w2_ref, b2_ref, out_ref, psum_ref, cnt_ref):
        _, y = _layer_y(aggb_ref, disr_ref, w_ref, b_ref, g_ref, be_ref, 32)
        bt = bt_ref[0]
        gi = lax.broadcasted_iota(jnp.int32, (G, RBLK), 0)
        one_t = (gi == bt).astype(jnp.float32)

        @pl.when(pl.program_id(0) == 0)
        def _():
            psum_ref[...] = jnp.zeros((G, 64), jnp.float32)
            cnt_ref[...] = jnp.zeros((G, 1), jnp.float32)

        psum_ref[...] += jnp.dot(one_t, y, preferred_element_type=jnp.float32)
        cnt_ref[...] += jnp.sum(one_t, axis=1, keepdims=True)

        @pl.when(pl.program_id(0) == nblk - 1)
        def _():
            pooled = psum_ref[...] / jnp.maximum(cnt_ref[...], 1.0)
            h = jnp.maximum(
                jnp.dot(pooled, w1_ref[...], preferred_element_type=jnp.float32)
                + b1_ref[...],
                0.0,
            )
            out_ref[...] = (
                jnp.sum(h * w2_ref[...], axis=1, keepdims=True) + b2_ref[...]
            )

    out, _, _ = pl.pallas_call(
        body,
        grid=(nblk,),
        in_specs=[
            pl.BlockSpec((2, RBLK, 32), lambda i: (0, i, 0)),
            pl.BlockSpec((RBLK, 16), lambda i: (i, 0)),
            pl.BlockSpec(w.shape, lambda i: (0, 0)),
        ] + _VEC_SPECS + [
            pl.BlockSpec((1, 1, RBLK), lambda i: (i, 0, 0)),
            pl.BlockSpec((64, 32), lambda i: (0, 0)),
            pl.BlockSpec((1, 32), lambda i: (0, 0)),
            pl.BlockSpec((1, 32), lambda i: (0, 0)),
            pl.BlockSpec((1, 1), lambda i: (0, 0)),
        ],
        out_specs=[
            pl.BlockSpec((G, 1), lambda i: (0, 0)),
            pl.BlockSpec((G, 64), lambda i: (0, 0)),
            pl.BlockSpec((G, 1), lambda i: (0, 0)),
        ],
        out_shape=[
            jax.ShapeDtypeStruct((G, 1), jnp.float32),
            jax.ShapeDtypeStruct((G, 64), jnp.float32),
            jax.ShapeDtypeStruct((G, 1), jnp.float32),
        ],
    )(aggb, disr, w, b, g, be, batch3, fc1w, fc1b, fc2v, fc2b)
    return out


def kernel(x, edge_index, batch, W0, b0, g0, be0, W1, b1, g1, be1,
           W2, b2, g2, be2, fc1_W, fc1_b, fc2_W, fc2_b):
    i32 = jnp.int32
    loop = jnp.arange(N, dtype=i32)
    src = jnp.concatenate([edge_index[0].astype(i32), loop])
    dst = jnp.concatenate([edge_index[1].astype(i32), loop])
    src_p = jnp.pad(src, (0, EP - ET), constant_values=N)
    dst_p = jnp.pad(dst, (0, EP - ET), constant_values=N)
    dst_ch = dst_p.reshape(NCH, CH)
    batch3 = jnp.pad(batch.astype(i32), (0, NP - N), constant_values=G).reshape(
        NP // RBLK, 1, RBLK
    )
    xp128 = jnp.zeros((NP, 16), jnp.float32).at[:N, :9].set(x).reshape(NP * 16 // 128, 128)
    W0p = jnp.zeros((16, 64), jnp.float32).at[:9, :].set(W0)

    degb = _sc_deg(dst_ch)
    t0, disr = _tc_prep(degb, xp128)
    agg0 = _sc_agg(t0, t0, src_p, dst_p, 16, feature_split=False, ub=8,
                   chw=128)
    t1a, t1b = _tc_layer(agg0, disr, W0p, b0.reshape(1, 64), g0.reshape(1, 64),
                         be0.reshape(1, 64), in_width=16)
    agg1 = _sc_agg(t1a, t1b, src_p, dst_p, 32, feature_split=True, ub=1,
                   chw=256)
    t2a, t2b = _tc_layer(agg1, disr, W1, b1.reshape(1, 64), g1.reshape(1, 64),
                         be1.reshape(1, 64), in_width=32)
    agg2 = _sc_agg(t2a, t2b, src_p, dst_p, 32, feature_split=True, ub=1,
                   chw=256)
    return _tc_layer_pool(agg2, disr, W2, b2.reshape(1, 64),
                          g2.reshape(1, 64), be2.reshape(1, 64), batch3,
                          fc1_W, fc1_b.reshape(1, 32), fc2_W.reshape(1, 32),
                          fc2_b.reshape(1, 1))
